# Initial kernel scaffold; baseline (speedup 1.0000x reference)
import jax
import jax.numpy as jnp
from jax.experimental import pallas as pl


def kernel(x, edge_index, eps, W1, b1, W2, b2):
    raise NotImplementedError("write your pallas kernel here")


# speedup vs baseline: 14.0146x; 14.0146x over previous
import functools

import jax
import jax.numpy as jnp
from jax import lax
from jax.experimental import pallas as pl
from jax.experimental.pallas import tpu as pltpu
from jax.experimental.pallas import tpu_sc as plsc

N, E, C, H = 10000, 320000, 128, 16

_info = plsc.get_sparse_core_info()
NC, NS = _info.num_cores, _info.num_subcores
NW = NC * NS
CH = 128
EC = -(-E // (NW * CH))
if EC % 2:
    EC += 1
E_PAD = NW * EC * CH
N_PAD = ((N + NS - 1) // NS + 15) // 16 * 16 * NS


def _mm1_body(x_ref, w_ref, y_ref):
    y_ref[...] = jnp.dot(x_ref[...], w_ref[...],
                         preferred_element_type=jnp.float32)


def _mlp_body(y_ref, p0_ref, p1_ref, s_ref, b1_ref, w2_ref, b2_ref, o_ref):
    scale = s_ref[0, 0]
    h = scale * y_ref[...] + p0_ref[...] + p1_ref[...] + b1_ref[...]
    h = jnp.maximum(h, 0.0)
    h = jnp.dot(h, w2_ref[...], preferred_element_type=jnp.float32) + b2_ref[...]
    o_ref[...] = jnp.maximum(h, 0.0)


def _make_scatter():
    mesh = plsc.VectorSubcoreMesh(core_axis_name="c", subcore_axis_name="s")

    @functools.partial(
        pl.kernel,
        mesh=mesh,
        out_type=jax.ShapeDtypeStruct((NC, N_PAD, H), jnp.float32),
        compiler_params=pltpu.CompilerParams(use_tc_tiling_on_sc=False),
        scratch_types=[
            pltpu.VMEM((EC, CH), jnp.int32),
            pltpu.VMEM((EC, CH), jnp.int32),
            pltpu.VMEM((CH, H), jnp.float32),
            pltpu.VMEM((CH, H), jnp.float32),
            pltpu.VMEM_SHARED((N_PAD, H), jnp.float32),
            pltpu.SemaphoreType.DMA,
            pltpu.SemaphoreType.DMA,
        ],
    )
    def scatter_k(src_hbm, dst_hbm, zeros_hbm, y_hbm, out_hbm,
                  src_v, dst_v, rows_a, rows_b, aggr, sem_a, sem_b):
        c = lax.axis_index("c")
        s = lax.axis_index("s")
        wid = s * NC + c

        pltpu.sync_copy(src_hbm.at[wid], src_v)
        pltpu.sync_copy(dst_hbm.at[wid], dst_v)

        @pl.when(s == 0)
        def _zero():
            pltpu.sync_copy(zeros_hbm, aggr)

        plsc.subcore_barrier()

        pltpu.async_copy(y_hbm.at[src_v.at[0]], rows_a, sem_a)

        def body(i, carry):
            j = 2 * i
            pltpu.async_copy(y_hbm.at[src_v.at[j + 1]], rows_b, sem_b)
            pltpu.make_async_copy(y_hbm.at[src_v.at[j]], rows_a, sem_a).wait()
            pltpu.sync_copy(rows_a, aggr.at[dst_v.at[j]], add=True)

            @pl.when(j + 2 < EC)
            def _fire_next():
                pltpu.async_copy(y_hbm.at[src_v.at[j + 2]], rows_a, sem_a)

            pltpu.make_async_copy(y_hbm.at[src_v.at[j + 1]], rows_b, sem_b).wait()
            pltpu.sync_copy(rows_b, aggr.at[dst_v.at[j + 1]], add=True)
            return carry

        lax.fori_loop(0, EC // 2, body, 0)

        plsc.subcore_barrier()

        rps = N_PAD // NS
        pltpu.sync_copy(aggr.at[pl.ds(s * rps, rps)],
                        out_hbm.at[c, pl.ds(s * rps, rps)])

    return scatter_k


_scatter_k = _make_scatter()


def kernel(x, edge_index, eps, W1, b1, W2, b2):
    assert x.shape == (N, C) and edge_index.shape == (2, E)

    y = pl.pallas_call(
        _mm1_body,
        out_shape=jax.ShapeDtypeStruct((N, H), jnp.float32),
    )(x, W1)

    src = edge_index[0]
    dst = edge_index[1]
    pad = E_PAD - E
    src3 = jnp.concatenate([src, jnp.zeros((pad,), jnp.int32)]).reshape(NW, EC, CH)
    dst3 = jnp.concatenate([dst, jnp.full((pad,), N, jnp.int32)]).reshape(NW, EC, CH)
    zeros = jnp.zeros((N_PAD, H), jnp.float32)

    partials = _scatter_k(src3, dst3, zeros, y)

    scale = (1.0 + eps).reshape(1, 1).astype(jnp.float32)
    out = pl.pallas_call(
        _mlp_body,
        out_shape=jax.ShapeDtypeStruct((N, H), jnp.float32),
    )(y, partials[0, :N], partials[1, :N], scale,
      b1.reshape(1, H), W2, b2.reshape(1, H))
    return out


# speedup vs baseline: 21.0915x; 1.5050x over previous
import functools

import jax
import jax.numpy as jnp
from jax import lax
from jax.experimental import pallas as pl
from jax.experimental.pallas import tpu as pltpu
from jax.experimental.pallas import tpu_sc as plsc

N, E, C, H = 10000, 320000, 128, 16

_info = plsc.get_sparse_core_info()
NC, NS = _info.num_cores, _info.num_subcores
NW = NC * NS
CH = 128
NCHUNK = E // CH
ECB = NCHUNK // NW
NEXTRA = NCHUNK - ECB * NW
RPS = N // NS
FR = N * H // 128


def _mm1_body(x_ref, w_ref, s_ref, y_ref, ys_ref):
    y = jnp.dot(x_ref[...], w_ref[...], preferred_element_type=jnp.float32)
    y_ref[...] = y
    ys_ref[...] = s_ref[0, 0] * y


def _mlp_body(p_ref, b1_ref, w2_ref, b2_ref, o_ref):
    h = p_ref[0] + p_ref[1] + b1_ref[...]
    h = jnp.maximum(h, 0.0)
    h = jnp.dot(h, w2_ref[...], preferred_element_type=jnp.float32) + b2_ref[...]
    o_ref[...] = jnp.maximum(h, 0.0)


def _make_scatter():
    mesh = plsc.VectorSubcoreMesh(core_axis_name="c", subcore_axis_name="s")

    @functools.partial(
        pl.kernel,
        mesh=mesh,
        out_type=jax.ShapeDtypeStruct((NC, N, H), jnp.float32),
        compiler_params=pltpu.CompilerParams(use_tc_tiling_on_sc=False),
        scratch_types=[
            pltpu.VMEM((ECB + 1, CH), jnp.int32),
            pltpu.VMEM((ECB + 1, CH), jnp.int32),
            pltpu.VMEM((CH, H), jnp.float32),
            pltpu.VMEM((CH, H), jnp.float32),
            pltpu.VMEM_SHARED((N, H), jnp.float32),
            pltpu.SemaphoreType.DMA,
            pltpu.SemaphoreType.DMA,
        ],
    )
    def scatter_k(ei_hbm, y_hbm, ys_hbm, out_hbm,
                  src_v, dst_v, rows_a, rows_b, aggr, sem_a, sem_b):
        c = lax.axis_index("c")
        s = lax.axis_index("s")
        wid = s * NC + c

        base = wid * ECB
        pltpu.sync_copy(ei_hbm.at[0, pl.ds(base, ECB)], src_v.at[pl.ds(0, ECB)])
        pltpu.sync_copy(ei_hbm.at[1, pl.ds(base, ECB)], dst_v.at[pl.ds(0, ECB)])

        @pl.when(wid >= NW - NEXTRA)
        def _extra_idx():
            xch = ECB * NW + (wid - (NW - NEXTRA))
            pltpu.sync_copy(ei_hbm.at[0, xch], src_v.at[ECB])
            pltpu.sync_copy(ei_hbm.at[1, xch], dst_v.at[ECB])

        @pl.when(c == 0)
        def _init_ys():
            pltpu.sync_copy(ys_hbm.at[pl.ds(s * RPS, RPS)],
                            aggr.at[pl.ds(s * RPS, RPS)])

        @pl.when(c == 1)
        def _init_zero():
            z = jnp.zeros((H,), jnp.float32)

            def zb(i, carry):
                rows_a[i, :] = z
                return carry

            lax.fori_loop(0, CH, zb, 0)
            nfull = RPS // CH
            for blk in range(nfull):
                pltpu.sync_copy(rows_a, aggr.at[pl.ds(s * RPS + blk * CH, CH)])
            rem = RPS - nfull * CH
            pltpu.sync_copy(rows_a.at[pl.ds(0, rem)],
                            aggr.at[pl.ds(s * RPS + nfull * CH, rem)])

        plsc.subcore_barrier()

        @pl.when(wid >= NW - NEXTRA)
        def _extra_chunk():
            pltpu.async_copy(y_hbm.at[src_v.at[ECB]], rows_a, sem_a).wait()
            pltpu.sync_copy(rows_a, aggr.at[dst_v.at[ECB]], add=True)

        pltpu.async_copy(y_hbm.at[src_v.at[0]], rows_a, sem_a)

        def body(i, carry):
            j = 2 * i
            pltpu.async_copy(y_hbm.at[src_v.at[j + 1]], rows_b, sem_b)
            pltpu.make_async_copy(y_hbm.at[src_v.at[j]], rows_a, sem_a).wait()
            pltpu.sync_copy(rows_a, aggr.at[dst_v.at[j]], add=True)

            @pl.when(j + 2 < ECB)
            def _fire_next():
                pltpu.async_copy(y_hbm.at[src_v.at[j + 2]], rows_a, sem_a)

            pltpu.make_async_copy(y_hbm.at[src_v.at[j + 1]], rows_b, sem_b).wait()
            pltpu.sync_copy(rows_b, aggr.at[dst_v.at[j + 1]], add=True)
            return carry

        lax.fori_loop(0, ECB // 2, body, 0)

        plsc.subcore_barrier()

        pltpu.sync_copy(aggr.at[pl.ds(s * RPS, RPS)],
                        out_hbm.at[c, pl.ds(s * RPS, RPS)])

    return scatter_k


_scatter_k = _make_scatter()


def kernel(x, edge_index, eps, W1, b1, W2, b2):
    assert x.shape == (N, C) and edge_index.shape == (2, E)

    scale = (1.0 + eps).reshape(1, 1).astype(jnp.float32)
    y, ys = pl.pallas_call(
        _mm1_body,
        out_shape=(jax.ShapeDtypeStruct((N, H), jnp.float32),
                   jax.ShapeDtypeStruct((N, H), jnp.float32)),
    )(x, W1, scale)

    ei3 = edge_index.reshape(2, NCHUNK, CH)
    partials = _scatter_k(ei3, y, ys)

    p_flat = partials.reshape(NC, FR, 128)
    b1t = jnp.tile(b1, 128 // H).reshape(1, 128)
    b2t = jnp.tile(b2, 128 // H).reshape(1, 128)
    w2bd = jnp.kron(jnp.eye(128 // H, dtype=jnp.float32), W2)
    o_flat = pl.pallas_call(
        _mlp_body,
        out_shape=jax.ShapeDtypeStruct((FR, 128), jnp.float32),
    )(p_flat, b1t, w2bd, b2t)
    return o_flat.reshape(N, H)


# speedup vs baseline: 24.8232x; 1.1769x over previous
import functools

import jax
import jax.numpy as jnp
from jax import lax
from jax.experimental import pallas as pl
from jax.experimental.pallas import tpu as pltpu
from jax.experimental.pallas import tpu_sc as plsc

N, E, C, H = 10000, 320000, 128, 16

_info = plsc.get_sparse_core_info()
NC, NS = _info.num_cores, _info.num_subcores
NW = NC * NS
CH = 128
NCHUNK = E // CH
ECB = NCHUNK // NW
NEXTRA = NCHUNK - ECB * NW
RPS = N // NS
FR = N * H // 128
NBUF = 3


def _mm1_body(x_ref, w_ref, y_ref):
    y_ref[...] = jnp.dot(x_ref[...], w_ref[...],
                         preferred_element_type=jnp.float32)


def _mlp_body(y_ref, p_ref, s_ref, b1_ref, w2_ref, b2_ref, o_ref):
    h = s_ref[0, 0] * y_ref[...] + p_ref[0] + p_ref[1] + b1_ref[...]
    h = jnp.maximum(h, 0.0)
    h = jnp.dot(h, w2_ref[...], preferred_element_type=jnp.float32) + b2_ref[...]
    o_ref[...] = jnp.maximum(h, 0.0)


def _make_scatter():
    mesh = plsc.VectorSubcoreMesh(core_axis_name="c", subcore_axis_name="s")

    @functools.partial(
        pl.kernel,
        mesh=mesh,
        out_type=jax.ShapeDtypeStruct((NC, N, H), jnp.float32),
        compiler_params=pltpu.CompilerParams(use_tc_tiling_on_sc=False),
        scratch_types=[
            pltpu.VMEM((ECB + 1, CH), jnp.int32),
            pltpu.VMEM((ECB + 1, CH), jnp.int32),
            [pltpu.VMEM((CH, H), jnp.float32)] * NBUF,
            pltpu.VMEM_SHARED((N, H), jnp.float32),
            [pltpu.SemaphoreType.DMA] * NBUF,
            [pltpu.SemaphoreType.DMA] * NBUF,
        ],
    )
    def scatter_k(ei_hbm, y_hbm, out_hbm,
                  src_v, dst_v, rows, aggr, gsem, ssem):
        c = lax.axis_index("c")
        s = lax.axis_index("s")
        wid = s * NC + c

        base = wid * ECB
        pltpu.sync_copy(ei_hbm.at[0, pl.ds(base, ECB)], src_v.at[pl.ds(0, ECB)])
        pltpu.sync_copy(ei_hbm.at[1, pl.ds(base, ECB)], dst_v.at[pl.ds(0, ECB)])

        @pl.when(wid >= NW - NEXTRA)
        def _extra_idx():
            xch = ECB * NW + (wid - (NW - NEXTRA))
            pltpu.sync_copy(ei_hbm.at[0, xch], src_v.at[ECB])
            pltpu.sync_copy(ei_hbm.at[1, xch], dst_v.at[ECB])

        z = jnp.zeros((H,), jnp.float32)

        def zb(i, carry):
            rows[0][i, :] = z
            return carry

        lax.fori_loop(0, CH, zb, 0)
        nfull = RPS // CH
        for blk in range(nfull):
            pltpu.sync_copy(rows[0], aggr.at[pl.ds(s * RPS + blk * CH, CH)])
        rem = RPS - nfull * CH
        pltpu.sync_copy(rows[0].at[pl.ds(0, rem)],
                        aggr.at[pl.ds(s * RPS + nfull * CH, rem)])

        plsc.subcore_barrier()

        @pl.when(wid >= NW - NEXTRA)
        def _extra_chunk():
            pltpu.async_copy(y_hbm.at[src_v.at[ECB]], rows[0], gsem[0]).wait()
            pltpu.sync_copy(rows[0], aggr.at[dst_v.at[ECB]], add=True)

        for b in range(NBUF):
            pltpu.async_copy(y_hbm.at[src_v.at[b]], rows[b], gsem[b])

        def body(i, carry):
            j0 = NBUF * i
            for b in range(NBUF):
                j = j0 + b
                pltpu.make_async_copy(
                    y_hbm.at[src_v.at[j]], rows[b], gsem[b]).wait()
                pltpu.async_copy(rows[b], aggr.at[dst_v.at[j]], ssem[b],
                                 add=True)
            for b in range(NBUF):
                j = j0 + b
                pltpu.make_async_copy(
                    rows[b], aggr.at[dst_v.at[j]], ssem[b]).wait()

                @pl.when(j + NBUF < ECB)
                def _fire_next():
                    pltpu.async_copy(y_hbm.at[src_v.at[j + NBUF]],
                                     rows[b], gsem[b])
            return carry

        lax.fori_loop(0, ECB // NBUF, body, 0)

        plsc.subcore_barrier()

        pltpu.sync_copy(aggr.at[pl.ds(s * RPS, RPS)],
                        out_hbm.at[c, pl.ds(s * RPS, RPS)])

    return scatter_k


_scatter_k = _make_scatter()


def kernel(x, edge_index, eps, W1, b1, W2, b2):
    assert x.shape == (N, C) and edge_index.shape == (2, E)

    y = pl.pallas_call(
        _mm1_body,
        out_shape=jax.ShapeDtypeStruct((N, H), jnp.float32),
    )(x, W1)

    ei3 = edge_index.reshape(2, NCHUNK, CH)
    partials = _scatter_k(ei3, y)

    p_flat = partials.reshape(NC, FR, 128)
    scale = (1.0 + eps).reshape(1, 1).astype(jnp.float32)
    b1t = jnp.tile(b1, 128 // H).reshape(1, 128)
    b2t = jnp.tile(b2, 128 // H).reshape(1, 128)
    w2bd = jnp.kron(jnp.eye(128 // H, dtype=jnp.float32), W2)
    o_flat = pl.pallas_call(
        _mlp_body,
        out_shape=jax.ShapeDtypeStruct((FR, 128), jnp.float32),
    )(y.reshape(FR, 128), p_flat, scale, b1t, w2bd, b2t)
    return o_flat.reshape(N, H)


# speedup vs baseline: 27.5974x; 1.1118x over previous
import functools

import jax
import jax.numpy as jnp
from jax import lax
from jax.experimental import pallas as pl
from jax.experimental.pallas import tpu as pltpu
from jax.experimental.pallas import tpu_sc as plsc

N, E, C, H = 10000, 320000, 128, 16

_info = plsc.get_sparse_core_info()
NC, NS = _info.num_cores, _info.num_subcores
NW = NC * NS
CH = 128
NCHUNK = E // CH
ECB = NCHUNK // NW
NEXTRA = NCHUNK - ECB * NW
RPS = N // NS
FR = N * H // 128
CB = 6
NG = ECB // CB
NGM = NG - 1
GR = CB * CH
NBUF = 3


def _mm1_body(x_ref, w_ref, y_ref):
    y_ref[...] = jnp.dot(x_ref[...], w_ref[...],
                         preferred_element_type=jnp.float32)


def _mlp_body(y_ref, p_ref, s_ref, b1_ref, w2_ref, b2_ref, o_ref):
    h = s_ref[0, 0] * y_ref[...] + p_ref[0] + p_ref[1] + b1_ref[...]
    h = jnp.maximum(h, 0.0)
    h = jnp.dot(h, w2_ref[...], preferred_element_type=jnp.float32) + b2_ref[...]
    o_ref[...] = jnp.maximum(h, 0.0)


def _make_scatter():
    mesh = plsc.VectorSubcoreMesh(core_axis_name="c", subcore_axis_name="s")

    @functools.partial(
        pl.kernel,
        mesh=mesh,
        out_type=jax.ShapeDtypeStruct((NC, N, H), jnp.float32),
        compiler_params=pltpu.CompilerParams(use_tc_tiling_on_sc=False),
        scratch_types=[
            pltpu.VMEM((ECB * CH + CH,), jnp.int32),
            pltpu.VMEM((ECB * CH + CH,), jnp.int32),
            [pltpu.VMEM((GR, H), jnp.float32)] * NBUF,
            pltpu.VMEM_SHARED((N, H), jnp.float32),
            [pltpu.SemaphoreType.DMA] * NBUF,
            [pltpu.SemaphoreType.DMA] * NBUF,
        ],
    )
    def scatter_k(ei_hbm, y_hbm, out_hbm,
                  src_v, dst_v, rows, aggr, gsem, ssem):
        c = lax.axis_index("c")
        s = lax.axis_index("s")
        wid = s * NC + c

        epb = ECB * CH
        base = wid * epb
        pltpu.sync_copy(ei_hbm.at[0, pl.ds(base, epb)], src_v.at[pl.ds(0, epb)])
        pltpu.sync_copy(ei_hbm.at[1, pl.ds(base, epb)], dst_v.at[pl.ds(0, epb)])

        @pl.when(wid >= NW - NEXTRA)
        def _extra_idx():
            xb = epb * NW + (wid - (NW - NEXTRA)) * CH
            pltpu.sync_copy(ei_hbm.at[0, pl.ds(xb, CH)],
                            src_v.at[pl.ds(epb, CH)])
            pltpu.sync_copy(ei_hbm.at[1, pl.ds(xb, CH)],
                            dst_v.at[pl.ds(epb, CH)])

        z = jnp.zeros((H,), jnp.float32)

        def zb(i, carry):
            rows[0][i, :] = z
            return carry

        lax.fori_loop(0, RPS, zb, 0)
        pltpu.sync_copy(rows[0].at[pl.ds(0, RPS)],
                        aggr.at[pl.ds(s * RPS, RPS)])

        plsc.subcore_barrier()

        @pl.when(wid >= NW - NEXTRA)
        def _extra_chunk():
            pltpu.async_copy(y_hbm.at[src_v.at[pl.ds(epb, CH)]],
                             rows[0].at[pl.ds(0, CH)], gsem[0]).wait()
            pltpu.sync_copy(rows[0].at[pl.ds(0, CH)],
                            aggr.at[dst_v.at[pl.ds(epb, CH)]], add=True)

        for b in range(NBUF):
            pltpu.async_copy(y_hbm.at[src_v.at[pl.ds(b * GR, GR)]],
                             rows[b], gsem[b])

        def body(i, carry):
            g0 = NBUF * i
            for b in range(NBUF):
                g = g0 + b
                pltpu.make_async_copy(
                    y_hbm.at[src_v.at[pl.ds(g * GR, GR)]],
                    rows[b], gsem[b]).wait()
                pltpu.async_copy(rows[b],
                                 aggr.at[dst_v.at[pl.ds(g * GR, GR)]],
                                 ssem[b], add=True)
            for b in range(NBUF):
                g = g0 + b
                pltpu.make_async_copy(
                    rows[b], aggr.at[dst_v.at[pl.ds(g * GR, GR)]],
                    ssem[b]).wait()

                @pl.when(g + NBUF < NGM)
                def _fire_next():
                    pltpu.async_copy(
                        y_hbm.at[src_v.at[pl.ds((g + NBUF) * GR, GR)]],
                        rows[b], gsem[b])
            return carry

        lax.fori_loop(0, NGM // NBUF, body, 0)

        pltpu.async_copy(y_hbm.at[src_v.at[pl.ds(NGM * GR, GR)]],
                         rows[0], gsem[0]).wait()
        pltpu.sync_copy(rows[0], aggr.at[dst_v.at[pl.ds(NGM * GR, GR)]],
                        add=True)

        plsc.subcore_barrier()

        pltpu.sync_copy(aggr.at[pl.ds(s * RPS, RPS)],
                        out_hbm.at[c, pl.ds(s * RPS, RPS)])

    return scatter_k


_scatter_k = _make_scatter()


def kernel(x, edge_index, eps, W1, b1, W2, b2):
    assert x.shape == (N, C) and edge_index.shape == (2, E)

    y = pl.pallas_call(
        _mm1_body,
        out_shape=jax.ShapeDtypeStruct((N, H), jnp.float32),
    )(x, W1)

    partials = _scatter_k(edge_index, y)

    p_flat = partials.reshape(NC, FR, 128)
    scale = (1.0 + eps).reshape(1, 1).astype(jnp.float32)
    b1t = jnp.tile(b1, 128 // H).reshape(1, 128)
    b2t = jnp.tile(b2, 128 // H).reshape(1, 128)
    w2bd = jnp.kron(jnp.eye(128 // H, dtype=jnp.float32), W2)
    o_flat = pl.pallas_call(
        _mlp_body,
        out_shape=jax.ShapeDtypeStruct((FR, 128), jnp.float32),
    )(y.reshape(FR, 128), p_flat, scale, b1t, w2bd, b2t)
    return o_flat.reshape(N, H)
